# Initial kernel scaffold; baseline (speedup 1.0000x reference)
import jax
import jax.numpy as jnp
from jax.experimental import pallas as pl


def kernel(queries, keys, values, planes):
    raise NotImplementedError("write your pallas kernel here")


# speedup vs baseline: 20.5287x; 20.5287x over previous
import jax
import jax.numpy as jnp
import numpy as np
from jax.experimental import pallas as pl
from jax.experimental.pallas import tpu as pltpu

C = 128
ITERS = 10
TOPK = 32

_INTERPRET = False


def _mm(a, b):
    return jnp.dot(a, b, preferred_element_type=jnp.float32)


def _mm_t0(a, b):
    return jax.lax.dot_general(a, b, (((0,), (0,)), ((), ())),
                               preferred_element_type=jnp.float32)


def _mm_t1(a, b):
    return jax.lax.dot_general(a, b, (((1,), (1,)), ((), ())),
                               preferred_element_type=jnp.float32)


def _split3(x):
    hi = x.astype(jnp.bfloat16).astype(jnp.float32)
    r = x - hi
    mid = r.astype(jnp.bfloat16).astype(jnp.float32)
    return hi, mid, r - mid


def _mm_pick(a, bsplit):
    bh, bm, bl = bsplit
    return (_mm(a, bh) + _mm(a, bm)) + _mm(a, bl)


def _assign_oh(bits, cent, c2r):
    L = bits.shape[0]
    x2 = jnp.sum(bits, axis=1, keepdims=True)
    dist = (x2 + c2r) - 2.0 * _mm_t1(bits, cent)
    md = jnp.min(dist, axis=1, keepdims=True)
    iota_c = jax.lax.broadcasted_iota(jnp.int32, (L, C), 1)
    cand = jnp.where(dist == md, iota_c, C)
    a2 = jnp.min(cand, axis=1, keepdims=True)
    return (iota_c == a2).astype(jnp.float32)


def _bits_body(q_ref, pw_ref, pb_ref, ioh_ref, bits_ref, cent0_ref):
    proj = _mm(q_ref[0], pw_ref[...]) + pb_ref[...]
    bits = (proj > 0).astype(jnp.float32)
    bits_ref[0] = bits
    cent0_ref[0] = _mm(ioh_ref[...], bits)


def _step_body(bits_ref, cent_ref, c2r_ref, out_ref):
    bits = bits_ref[0]
    cent = cent_ref[0]
    L = bits.shape[0]
    oh = _assign_oh(bits, cent, c2r_ref[0])
    cnt = _mm_t0(oh, jnp.ones((L, 1), jnp.float32))
    sums = _mm_t0(oh, bits)
    new = sums / jnp.maximum(cnt, 1.0)
    ind = (cnt > 0).astype(jnp.float32)
    out_ref[0] = new * ind + cent * (1.0 - ind)


def _attn_body(q_ref, k_ref, v_ref, bits_ref, cent_ref, c2r_ref, o_ref,
               kc_ref, vc_ref):
    f32 = jnp.float32
    qh = q_ref[0]
    kh = k_ref[0]
    vh = v_ref[0]
    L, E = qh.shape
    S = kh.shape[0]
    D = vh.shape[1]
    temp = 1.0 / float(np.sqrt(E))

    oh = _assign_oh(bits_ref[0], cent_ref[0], c2r_ref[0])
    cnt = _mm_t0(oh, jnp.ones((L, 1), f32))

    qg = _mm_t0(oh, qh) / jnp.maximum(cnt, 1.0)
    QK = _mm_t1(qg, kh)

    st = temp * QK
    m = jnp.max(st, axis=1, keepdims=True)
    ex = jnp.exp(st - m)
    A_full = ex / jnp.sum(ex, axis=1, keepdims=True)

    iota_s = jax.lax.broadcasted_iota(jnp.int32, (C, S), 1)
    kh_split = _split3(kh)
    vh_split = _split3(vh)

    def topk_step(kk, carry):
        QKm, chosen = carry
        mx = jnp.max(QKm, axis=1, keepdims=True)
        cand = jnp.where(QKm == mx, iota_s, S)
        am2 = jnp.min(cand, axis=1, keepdims=True)
        ohk = iota_s == am2
        ohkf = ohk.astype(f32)
        kc_ref[kk] = _mm_pick(ohkf, kh_split)
        vc_ref[kk] = _mm_pick(ohkf, vh_split)
        return jnp.where(ohk, -jnp.inf, QKm), chosen + ohkf

    QKm, chosen = jax.lax.fori_loop(
        0, TOPK, topk_step, (QK, jnp.zeros((C, S), f32)))

    A_bot = A_full * (1.0 - chosen)
    A_bk_c = jnp.sum(A_bot, axis=1, keepdims=True)
    V_bot_c = _mm(A_bot, vh)

    iota_k = jax.lax.broadcasted_iota(jnp.int32, (L, TOPK), 1)

    def qkt_step(kk, QKt):
        Kq = _mm_pick(oh, _split3(kc_ref[kk]))
        col = jnp.sum(qh * Kq, axis=1, keepdims=True)
        return QKt + col * (iota_k == kk).astype(f32)

    QKt = jax.lax.fori_loop(0, TOPK, qkt_step, jnp.zeros((L, TOPK), f32))

    stt = temp * QKt
    mt = jnp.max(stt, axis=1, keepdims=True)
    ext = jnp.exp(stt - mt)
    A_top = ext / jnp.sum(ext, axis=1, keepdims=True)
    A_bk_q = _mm_pick(oh, _split3(A_bk_c))
    A_top = A_top * (1.0 - A_bk_q)

    def vtop_step(kk, V_top):
        Vq = _mm_pick(oh, _split3(vc_ref[kk]))
        w = jnp.sum(A_top * (iota_k == kk).astype(f32),
                    axis=1, keepdims=True)
        return V_top + w * Vq

    V_top = jax.lax.fori_loop(0, TOPK, vtop_step, jnp.zeros((L, D), f32))

    V_bottom = _mm_pick(oh, _split3(V_bot_c))
    o_ref[0] = V_top + V_bottom


def kernel(queries, keys, values, planes):
    N, L, H, E = queries.shape
    S = keys.shape[1]
    D = values.shape[3]
    BITS = planes.shape[0]
    NH = N * H
    f32 = jnp.float32

    q3 = jnp.transpose(queries, (0, 2, 1, 3)).reshape(NH, L, E)
    k3 = jnp.transpose(keys, (0, 2, 1, 3)).reshape(NH, S, E)
    v3 = jnp.transpose(values, (0, 2, 1, 3)).reshape(NH, S, D)
    pw = jnp.transpose(planes[:, :-1])
    pb = jnp.transpose(planes[:, -1:])
    init_idx = jnp.linspace(0, L - 1, C).astype(jnp.int32)
    init_oh = jax.nn.one_hot(init_idx, L, dtype=f32)

    head_spec = lambda shape: pl.BlockSpec((1,) + shape, lambda h: (h, 0, 0))
    full_spec = lambda arr: pl.BlockSpec(arr.shape, lambda h: (0,) * arr.ndim)

    bits, cent = pl.pallas_call(
        _bits_body,
        grid=(NH,),
        in_specs=[head_spec((L, E)), full_spec(pw), full_spec(pb),
                  full_spec(init_oh)],
        out_specs=[head_spec((L, BITS)), head_spec((C, BITS))],
        out_shape=[jax.ShapeDtypeStruct((NH, L, BITS), f32),
                   jax.ShapeDtypeStruct((NH, C, BITS), f32)],
        interpret=_INTERPRET,
    )(q3, pw, pb, init_oh)

    step_call = pl.pallas_call(
        _step_body,
        grid=(NH,),
        in_specs=[head_spec((L, BITS)), head_spec((C, BITS)),
                  head_spec((1, C))],
        out_specs=head_spec((C, BITS)),
        out_shape=jax.ShapeDtypeStruct((NH, C, BITS), f32),
        interpret=_INTERPRET,
    )

    for _ in range(ITERS):
        c2r = jnp.sum(cent * cent, axis=-1)[:, None, :]
        cent = step_call(bits, cent, c2r)

    c2r = jnp.sum(cent * cent, axis=-1)[:, None, :]

    out = pl.pallas_call(
        _attn_body,
        grid=(NH,),
        in_specs=[head_spec((L, E)), head_spec((S, E)), head_spec((S, D)),
                  head_spec((L, BITS)), head_spec((C, BITS)),
                  head_spec((1, C))],
        out_specs=head_spec((L, D)),
        out_shape=jax.ShapeDtypeStruct((NH, L, D), f32),
        scratch_shapes=[
            pltpu.VMEM((TOPK, C, E), f32),
            pltpu.VMEM((TOPK, C, D), f32),
        ],
        interpret=_INTERPRET,
    )(q3, k3, v3, bits, cent, c2r)

    return jnp.transpose(out.reshape(N, H, L, D), (0, 2, 1, 3))


# speedup vs baseline: 25.4816x; 1.2413x over previous
import jax
import jax.numpy as jnp
import numpy as np
from jax.experimental import pallas as pl
from jax.experimental.pallas import tpu as pltpu

C = 128
ITERS = 10
TOPK = 32


def _mm(a, b):
    return jnp.dot(a, b, preferred_element_type=jnp.float32)


def _mm_t0(a, b):
    return jax.lax.dot_general(a, b, (((0,), (0,)), ((), ())),
                               preferred_element_type=jnp.float32)


def _mm_t1(a, b):
    return jax.lax.dot_general(a, b, (((1,), (1,)), ((), ())),
                               preferred_element_type=jnp.float32)


def _split3(x):
    hi = x.astype(jnp.bfloat16).astype(jnp.float32)
    r = x - hi
    mid = r.astype(jnp.bfloat16).astype(jnp.float32)
    return hi, mid, r - mid


def _split2(x):
    hi = x.astype(jnp.bfloat16).astype(jnp.float32)
    mid = (x - hi).astype(jnp.bfloat16).astype(jnp.float32)
    return hi, mid


def _mm_pick(a, bsplit):
    bh, bm, bl = bsplit
    return (_mm(a, bh) + _mm(a, bm)) + _mm(a, bl)


def _assign_oh(bits, cent, c2r):
    L = bits.shape[0]
    x2 = jnp.sum(bits, axis=1, keepdims=True)
    dist = (x2 + c2r) - 2.0 * _mm_t1(bits, cent)
    md = jnp.min(dist, axis=1, keepdims=True)
    iota_c = jax.lax.broadcasted_iota(jnp.int32, (L, C), 1)
    cand = jnp.where(dist == md, iota_c, C)
    a2 = jnp.min(cand, axis=1, keepdims=True)
    return (iota_c == a2).astype(jnp.float32)


def _bits_body(q_ref, pw_ref, pb_ref, ioh_ref, bits_ref, cent0_ref):
    proj = _mm(q_ref[0], pw_ref[...]) + pb_ref[...]
    bits = (proj > 0).astype(jnp.float32)
    bits_ref[0] = bits
    cent0_ref[0] = _mm(ioh_ref[...], bits)


def _step_body(bits_ref, cent_ref, c2r_ref, out_ref):
    NH = bits_ref.shape[0]
    for h in range(NH):
        bits = bits_ref[h]
        cent = cent_ref[h]
        L = bits.shape[0]
        oh = _assign_oh(bits, cent, c2r_ref[h])
        cnt = _mm_t0(oh, jnp.ones((L, 1), jnp.float32))
        sums = _mm_t0(oh, bits)
        new = sums / jnp.maximum(cnt, 1.0)
        ind = (cnt > 0).astype(jnp.float32)
        out_ref[h] = new * ind + cent * (1.0 - ind)


def _attn_body(q_ref, k_ref, v_ref, bits_ref, cent_ref, c2r_ref, o_ref,
               kvc_hi_ref, kvc_mid_ref):
    f32 = jnp.float32
    qh = q_ref[0]
    kh = k_ref[0]
    vh = v_ref[0]
    L, E = qh.shape
    S = kh.shape[0]
    D = vh.shape[1]
    temp = 1.0 / float(np.sqrt(E))

    oh = _assign_oh(bits_ref[0], cent_ref[0], c2r_ref[0])
    cnt = _mm_t0(oh, jnp.ones((L, 1), f32))

    qg = _mm_t0(oh, qh) / jnp.maximum(cnt, 1.0)
    QK = _mm_t1(qg, kh)

    st = temp * QK
    m = jnp.max(st, axis=1, keepdims=True)
    ex = jnp.exp(st - m)
    A_full = ex / jnp.sum(ex, axis=1, keepdims=True)

    iota_s = jax.lax.broadcasted_iota(jnp.int32, (C, S), 1)
    kh_hi, kh_mid = _split2(kh)
    vh_hi, vh_mid = _split2(vh)
    khv_hi = jnp.concatenate([kh_hi, vh_hi], axis=1)
    khv_mid = jnp.concatenate([kh_mid, vh_mid], axis=1)

    def topk_step(kk, carry):
        QKm, chosen = carry
        mx = jnp.max(QKm, axis=1, keepdims=True)
        cand = jnp.where(QKm == mx, iota_s, S)
        am2 = jnp.min(cand, axis=1, keepdims=True)
        ohk = iota_s == am2
        ohkf = ohk.astype(f32)
        kvc_hi_ref[kk] = _mm(ohkf, khv_hi)
        kvc_mid_ref[kk] = _mm(ohkf, khv_mid)
        return jnp.where(ohk, -jnp.inf, QKm), chosen + ohkf

    QKm, chosen = jax.lax.fori_loop(
        0, TOPK, topk_step, (QK, jnp.zeros((C, S), f32)))

    A_bot = A_full * (1.0 - chosen)
    A_bk_c = jnp.sum(A_bot, axis=1, keepdims=True)
    V_bot_c = _mm(A_bot, vh)

    iota_k = jax.lax.broadcasted_iota(jnp.int32, (L, TOPK), 1)

    def qkt_step(kk, QKt):
        Kq = (_mm(oh, kvc_hi_ref[kk][:, :E])
              + _mm(oh, kvc_mid_ref[kk][:, :E]))
        col = jnp.sum(qh * Kq, axis=1, keepdims=True)
        return QKt + col * (iota_k == kk).astype(f32)

    QKt = jax.lax.fori_loop(0, TOPK, qkt_step, jnp.zeros((L, TOPK), f32))

    stt = temp * QKt
    mt = jnp.max(stt, axis=1, keepdims=True)
    ext = jnp.exp(stt - mt)
    A_top = ext / jnp.sum(ext, axis=1, keepdims=True)
    A_bk_q = _mm_pick(oh, _split3(A_bk_c))
    A_top = A_top * (1.0 - A_bk_q)

    def vtop_step(kk, V_top):
        Vq = (_mm(oh, kvc_hi_ref[kk][:, E:])
              + _mm(oh, kvc_mid_ref[kk][:, E:]))
        w = jnp.sum(A_top * (iota_k == kk).astype(f32),
                    axis=1, keepdims=True)
        return V_top + w * Vq

    V_top = jax.lax.fori_loop(0, TOPK, vtop_step, jnp.zeros((L, D), f32))

    V_bottom = _mm_pick(oh, _split3(V_bot_c))
    o_ref[0] = V_top + V_bottom


def kernel(queries, keys, values, planes):
    N, L, H, E = queries.shape
    S = keys.shape[1]
    D = values.shape[3]
    BITS = planes.shape[0]
    NH = N * H
    f32 = jnp.float32

    q3 = jnp.transpose(queries, (0, 2, 1, 3)).reshape(NH, L, E)
    k3 = jnp.transpose(keys, (0, 2, 1, 3)).reshape(NH, S, E)
    v3 = jnp.transpose(values, (0, 2, 1, 3)).reshape(NH, S, D)
    pw = jnp.transpose(planes[:, :-1])
    pb = jnp.transpose(planes[:, -1:])
    init_idx = jnp.linspace(0, L - 1, C).astype(jnp.int32)
    init_oh = jax.nn.one_hot(init_idx, L, dtype=f32)

    head_spec = lambda shape: pl.BlockSpec((1,) + shape, lambda h: (h, 0, 0))
    full_spec = lambda arr: pl.BlockSpec(arr.shape, lambda h: (0,) * arr.ndim)

    bits, cent = pl.pallas_call(
        _bits_body,
        grid=(NH,),
        in_specs=[head_spec((L, E)), full_spec(pw), full_spec(pb),
                  full_spec(init_oh)],
        out_specs=[head_spec((L, BITS)), head_spec((C, BITS))],
        out_shape=[jax.ShapeDtypeStruct((NH, L, BITS), f32),
                   jax.ShapeDtypeStruct((NH, C, BITS), f32)],
    )(q3, pw, pb, init_oh)

    step_call = pl.pallas_call(
        _step_body,
        in_specs=[pl.BlockSpec((NH, L, BITS), lambda: (0, 0, 0)),
                  pl.BlockSpec((NH, C, BITS), lambda: (0, 0, 0)),
                  pl.BlockSpec((NH, 1, C), lambda: (0, 0, 0))],
        out_specs=pl.BlockSpec((NH, C, BITS), lambda: (0, 0, 0)),
        out_shape=jax.ShapeDtypeStruct((NH, C, BITS), f32),
    )

    for _ in range(ITERS):
        c2r = jnp.sum(cent * cent, axis=-1)[:, None, :]
        cent = step_call(bits, cent, c2r)

    c2r = jnp.sum(cent * cent, axis=-1)[:, None, :]

    out = pl.pallas_call(
        _attn_body,
        grid=(NH,),
        in_specs=[head_spec((L, E)), head_spec((S, E)), head_spec((S, D)),
                  head_spec((L, BITS)), head_spec((C, BITS)),
                  head_spec((1, C))],
        out_specs=head_spec((L, D)),
        out_shape=jax.ShapeDtypeStruct((NH, L, D), f32),
        scratch_shapes=[
            pltpu.VMEM((TOPK, C, E + D), f32),
            pltpu.VMEM((TOPK, C, E + D), f32),
        ],
    )(q3, k3, v3, bits, cent, c2r)

    return jnp.transpose(out.reshape(N, H, L, D), (0, 2, 1, 3))
